# baseline (speedup 1.0000x reference)
import jax
import jax.numpy as jnp
from jax.experimental import pallas as pl

N = 10000


def _median_aggregate(msgs, dst, num_nodes):
    counts = jnp.bincount(dst, length=num_nodes)
    starts = jnp.concatenate([jnp.zeros((1,), counts.dtype), jnp.cumsum(counts)[:-1]])

    def _col_sorted(v):
        order = jnp.lexsort((v, dst))
        return v[order]

    sorted_cols = jax.vmap(_col_sorted, in_axes=1, out_axes=1)(msgs)
    med_idx = (jnp.maximum(counts, 1) - 1) // 2
    gidx = starts + med_idx
    out = sorted_cols[gidx]
    out = jnp.where((counts > 0)[:, None], out, jnp.zeros_like(out))
    return out


def _median_conv(x, W, b, src, dst, num_nodes):
    h = x @ W
    msgs = h[src]
    out = _median_aggregate(msgs, dst, num_nodes)
    return out + b


def kernel(x, edge_index, W1, b1, W2, b2):
    loop = jnp.arange(N, dtype=edge_index.dtype)
    src = jnp.concatenate([edge_index[0], loop])
    dst = jnp.concatenate([edge_index[1], loop])
    h = _median_conv(x, W1, b1, src, dst, N)
    h = jax.nn.relu(h)
    h = _median_conv(h, W2, b2, src, dst, N)
    return jax.nn.log_softmax(h, axis=1)


# speedup vs baseline: 9.1621x; 9.1621x over previous
import functools

import numpy as np
import jax
import jax.numpy as jnp
from jax import lax
from jax.experimental import pallas as pl
from jax.experimental.pallas import tpu as pltpu
from jax.experimental.pallas import tpu_sc as plsc

N = 10000
E2 = 170000
NW = 32
NPT = 320
NPAD = NW * NPT
CHUNK = 2000
NCH = E2 // CHUNK
E2P = 84 * 2048
STG = 160
WCAP = 512
CSTEP = 504
MSBI = np.int32(-2147483648)
LOWI = np.int32(0x7FFFFFFF)

_MESH = plsc.VectorSubcoreMesh(core_axis_name="c", subcore_axis_name="s")


def _wid():
    return lax.axis_index("s") * 2 + lax.axis_index("c")


def _iota16():
    return lax.iota(jnp.int32, 16)


def _sread(ref, i):
    return ref[pl.ds(i, 16)][0]


@functools.partial(
    pl.kernel,
    out_type=[
        jax.ShapeDtypeStruct((NPAD,), jnp.int32),
        jax.ShapeDtypeStruct((NW, E2P), jnp.int32),
        jax.ShapeDtypeStruct((NW, E2P), jnp.int32),
    ],
    mesh=_MESH,
    compiler_params=pltpu.CompilerParams(needs_layout_passes=False),
    scratch_types=[
        pltpu.VMEM((CHUNK,), jnp.int32),
        pltpu.VMEM((CHUNK,), jnp.int32),
        pltpu.VMEM((STG,), jnp.int32),
        pltpu.VMEM((STG,), jnp.int32),
        pltpu.VMEM((NPT,), jnp.int32),
    ],
)
def _a1(dst_hbm, src_hbm, counts_hbm, scd_hbm, scs_hbm,
        dch, sch, sd, ss, cnts):
    wid = _wid()
    lo = wid * NPT
    z16 = jnp.zeros((16,), jnp.int32)

    def z1(i, _):
        cnts[pl.ds(i * 16, 16)] = z16
        return 0
    lax.fori_loop(0, NPT // 16, z1, 0)

    def flush128(fl):
        fo = pl.multiple_of(fl * 128, 8)
        pltpu.sync_copy(sd.at[pl.ds(0, 128)], scd_hbm.at[wid, pl.ds(fo, 128)])
        pltpu.sync_copy(ss.at[pl.ds(0, 128)], scs_hbm.at[wid, pl.ds(fo, 128)])
        td = sd[pl.ds(128, 16)]
        ts = ss[pl.ds(128, 16)]
        sd[pl.ds(0, 16)] = td
        ss[pl.ds(0, 16)] = ts

    def chunk_body(ci, carry):
        bp, fl = carry
        co = pl.multiple_of(ci * CHUNK, 8)
        pltpu.sync_copy(dst_hbm.at[pl.ds(co, CHUNK)], dch)
        pltpu.sync_copy(src_hbm.at[pl.ds(co, CHUNK)], sch)

        def g_body(g, carry):
            bp, fl = carry
            dv = dch[pl.ds(g * 16, 16)]
            sv = sch[pl.ds(g * 16, 16)]
            m = (dv >= lo) & (dv < lo + NPT)
            nn = jnp.clip(dv - lo, 0, NPT - 1)
            occ, lastm = plsc.scan_count(nn, mask=m)
            plsc.addupdate_scatter(cnts, [nn], occ, mask=m & lastm)
            plsc.store_compressed(sd.at[pl.ds(bp, 16)], dv, mask=m)
            plsc.store_compressed(ss.at[pl.ds(bp, 16)], sv, mask=m)
            bp = bp + jnp.sum(m.astype(jnp.int32))

            def do_flush(carry):
                bp, fl = carry
                flush128(fl)
                return bp - 128, fl + 1
            return lax.cond(bp >= 128, do_flush, lambda c: c, (bp, fl))
        return lax.fori_loop(0, CHUNK // 16, g_body, (bp, fl))
    bp, fl = lax.fori_loop(0, NCH, chunk_body, (0, 0))
    flush128(fl)
    flush128(fl + 1)
    pltpu.sync_copy(cnts, counts_hbm.at[pl.ds(lo, NPT)])


@functools.partial(
    pl.kernel,
    out_type=[
        jax.ShapeDtypeStruct((NPAD,), jnp.int32),
        jax.ShapeDtypeStruct((E2 + 8,), jnp.int32),
    ],
    mesh=_MESH,
    compiler_params=pltpu.CompilerParams(needs_layout_passes=False),
    scratch_types=[
        pltpu.VMEM((2048,), jnp.int32),
        pltpu.VMEM((2048,), jnp.int32),
        pltpu.VMEM((NPT,), jnp.int32),
        pltpu.VMEM((NPT,), jnp.int32),
        pltpu.VMEM((NPT,), jnp.int32),
        pltpu.VMEM((STG,), jnp.int32),
        pltpu.VMEM((STG,), jnp.int32),
        pltpu.VMEM((128,), jnp.int32),
        pltpu.VMEM((128,), jnp.int32),
        pltpu.SemaphoreType.DMA,
    ],
)
def _a2(counts_hbm, scd_hbm, scs_hbm, starts_hbm, srcs_hbm,
        dch, sch, cbuf, stb, run, sp, sv, posb, valb, sem):
    wid = _wid()
    lo = wid * NPT

    def pw(w2, base):
        pltpu.sync_copy(counts_hbm.at[pl.ds(pl.multiple_of(w2 * NPT, 8), NPT)], cbuf)

        def ps(t, b):
            return b + jnp.sum(cbuf[pl.ds(t * 16, 16)])
        return lax.fori_loop(0, NPT // 16, ps, base)
    base = lax.fori_loop(0, wid, pw, 0)

    pltpu.sync_copy(counts_hbm.at[pl.ds(lo, NPT)], cbuf)

    def sc_body(t, s):
        g = cbuf[pl.ds(t * 16, 16)]
        excl = plsc.cumsum(g) - g + s
        stb[pl.ds(t * 16, 16)] = excl
        run[pl.ds(t * 16, 16)] = excl
        return s + jnp.sum(g)
    total = lax.fori_loop(0, NPT // 16, sc_body, base)
    local_e = total - base
    pltpu.sync_copy(stb, starts_hbm.at[pl.ds(lo, NPT)])

    def flush128(fl):
        for q in range(8):
            posb[pl.ds(q * 16, 16)] = sp[pl.ds(q * 16, 16)]
            valb[pl.ds(q * 16, 16)] = sv[pl.ds(q * 16, 16)]
        pltpu.async_copy(valb, srcs_hbm.at[posb], sem).wait()
        tp = sp[pl.ds(128, 16)]
        tv = sv[pl.ds(128, 16)]
        sp[pl.ds(0, 16)] = tp
        sv[pl.ds(0, 16)] = tv

    nchk = (local_e + 2047) // 2048

    def chunk_body(ci, carry):
        bp = carry
        co = pl.multiple_of(ci * 2048, 8)
        pltpu.sync_copy(scd_hbm.at[wid, pl.ds(co, 2048)], dch)
        pltpu.sync_copy(scs_hbm.at[wid, pl.ds(co, 2048)], sch)

        def g_body(g, bp):
            dv = dch[pl.ds(g * 16, 16)]
            s_v = sch[pl.ds(g * 16, 16)]
            valid = (ci * 2048 + g * 16 + _iota16()) < local_e
            nn = jnp.clip(dv - lo, 0, NPT - 1)
            occ, lastm = plsc.scan_count(nn, mask=valid)
            bs = plsc.load_gather(run, [nn], mask=valid)
            pos = bs + occ - 1
            plsc.addupdate_scatter(run, [nn], occ, mask=valid & lastm)
            plsc.store_compressed(sp.at[pl.ds(bp, 16)], pos, mask=valid)
            plsc.store_compressed(sv.at[pl.ds(bp, 16)], s_v, mask=valid)
            bp = bp + jnp.sum(valid.astype(jnp.int32))

            def do_flush(bp):
                flush128(0)
                return bp - 128
            return lax.cond(bp >= 128, do_flush, lambda b: b, bp)
        return lax.fori_loop(0, 128, g_body, bp)
    bp = lax.fori_loop(0, nchk, chunk_body, 0)

    def pad_and_flush(bp):
        for q in range(STG // 16):
            lane = _iota16() + q * 16
            pv = sp[pl.ds(q * 16, 16)]
            sp[pl.ds(q * 16, 16)] = jnp.where(lane < bp, pv, E2)
        flush128(0)
        return jnp.maximum(bp - 128, 0)
    bp = pad_and_flush(bp)
    pad_and_flush(bp)


def _make_median(ngrp):
    C = ngrp * 16

    def gather_convert(h_hbm, srcs_hbm, idx4, rb, ub, sem, base_al):
        base_al = pl.multiple_of(base_al, 8)
        for kk in range(4):
            pltpu.sync_copy(srcs_hbm.at[pl.ds(base_al + kk * 128, 128)], idx4[kk])
        for kk in range(4):
            pltpu.async_copy(h_hbm.at[idx4[kk]], rb.at[pl.ds(kk * 128, 128)], sem).wait()

        def cv(e, _):
            for g in range(ngrp):
                v = rb[e, pl.ds(g * 16, 16)]
                b = plsc.bitcast(v, jnp.int32)
                m = b >> 31
                ub[e, pl.ds(g * 16, 16)] = b ^ (m & LOWI)
            return 0
        lax.fori_loop(0, WCAP, cv, 0)

    def select_update(P, cnts, Ts, k1):
        return tuple(
            jnp.where(cnts[g] >= k1, P[g], Ts[g]) for g in range(ngrp))

    def count_seg(ub, e0, d, Ts_cmp, cnts):
        def e_body(e, cn):
            row = e0 + e
            return tuple(
                cn[g] + (ub[row, pl.ds(g * 16, 16)] < Ts_cmp[g]).astype(jnp.int32)
                for g in range(ngrp))
        return lax.fori_loop(0, d, e_body, cnts)

    def finish(P, outb, i):
        for g in range(ngrp):
            ui = P[g]
            b = jnp.where(ui < 0, ui ^ MSBI, ~ui)
            outb[i, pl.ds(g * 16, 16)] = plsc.bitcast(b, jnp.float32)

    @functools.partial(
        pl.kernel,
        out_type=jax.ShapeDtypeStruct((NPAD, C), jnp.float32),
        mesh=_MESH,
        compiler_params=pltpu.CompilerParams(
            needs_layout_passes=False, use_tc_tiling_on_sc=False),
        scratch_types=[
            pltpu.VMEM((NPT + 16,), jnp.int32),
            pltpu.VMEM((NPT + 16,), jnp.int32),
            pltpu.VMEM((128,), jnp.int32),
            pltpu.VMEM((128,), jnp.int32),
            pltpu.VMEM((128,), jnp.int32),
            pltpu.VMEM((128,), jnp.int32),
            pltpu.VMEM((WCAP, C), jnp.float32),
            pltpu.VMEM((WCAP, C), jnp.int32),
            pltpu.VMEM((NPT, C), jnp.float32),
            pltpu.SemaphoreType.DMA,
        ],
    )
    def med(h_hbm, srcs_hbm, counts_hbm, starts_hbm, m_hbm,
            cbuf, stb, i0, i1, i2, i3, rb, ub, outb, sem):
        wid = _wid()
        lo = wid * NPT
        idx4 = (i0, i1, i2, i3)
        pltpu.sync_copy(counts_hbm.at[pl.ds(lo, NPT)], cbuf.at[pl.ds(0, NPT)])
        pltpu.sync_copy(starts_hbm.at[pl.ds(lo, NPT)], stb.at[pl.ds(0, NPT)])
        fz16 = jnp.zeros((16,), jnp.float32)

        def zb(i, _):
            for g in range(ngrp):
                outb[i, pl.ds(g * 16, 16)] = fz16
            return 0
        lax.fori_loop(0, NPT, zb, 0)

        zP = tuple(jnp.zeros((16,), jnp.int32) for _ in range(ngrp))
        zC = tuple(jnp.zeros((16,), jnp.int32) for _ in range(ngrp))

        def win_body(n):
            d = _sread(cbuf, n)
            ws = _sread(stb, n)
            ws_al = jnp.minimum(ws - lax.rem(ws, 8), E2 - WCAP)
            in_window = (ws + d) <= (ws_al + WCAP)

            def dowin(n):
                limit = ws_al + WCAP

                def ext_body(carry):
                    mm, _ = carry
                    ok = (mm < NPT) & (_sread(stb, mm) + _sread(cbuf, mm) <= limit)
                    return jnp.where(ok, mm + 1, mm), ~ok
                m, _ = lax.while_loop(lambda c: ~c[1], ext_body, (n + 1, False))
                gather_convert(h_hbm, srcs_hbm, idx4, rb, ub, sem, ws_al)

                def node_body(i, _):
                    di = _sread(cbuf, i)

                    def comp(_):
                        k1 = (di - 1) // 2 + 1
                        e0 = _sread(stb, i) - ws_al

                        def bit_body(t, P):
                            bitv = np.int32(1) << (31 - t)
                            Ts = tuple(P[g] | bitv for g in range(ngrp))
                            Tc = tuple(Ts[g] ^ MSBI for g in range(ngrp))
                            cnts = count_seg(ub, e0, di, Tc, zC)
                            return select_update(P, cnts, Ts, k1)
                        P = lax.fori_loop(0, 32, bit_body, zP)
                        finish(P, outb, i)
                        return 0
                    return lax.cond(di > 0, comp, lambda _: 0, 0)
                lax.fori_loop(n, m, node_body, 0)
                return m

            def dobig(n):
                k1 = (d - 1) // 2 + 1
                nchk = (d + CSTEP - 1) // CSTEP

                def bit_body(t, P):
                    bitv = np.int32(1) << (31 - t)
                    Ts = tuple(P[g] | bitv for g in range(ngrp))
                    Tc = tuple(Ts[g] ^ MSBI for g in range(ngrp))

                    def chunk_body(c2, cn):
                        es = ws + c2 * CSTEP
                        es_al = jnp.minimum(es - lax.rem(es, 8), E2 - WCAP)
                        gather_convert(h_hbm, srcs_hbm, idx4, rb, ub, sem, es_al)
                        cl = jnp.minimum(CSTEP, d - c2 * CSTEP)
                        return count_seg(ub, es - es_al, cl, Tc, cn)
                    cnts = lax.fori_loop(0, nchk, chunk_body, zC)
                    return select_update(P, cnts, Ts, k1)
                P = lax.fori_loop(0, 32, bit_body, zP)
                finish(P, outb, n)
                return n + 1

            def nonzero(n):
                return lax.cond(in_window, dowin, dobig, n)
            return lax.cond(d == 0, lambda v: v + 1, nonzero, n)
        lax.while_loop(lambda n: n < NPT, win_body, 0)
        pltpu.sync_copy(outb, m_hbm.at[pl.ds(lo, NPT)])

    return med


_med64 = _make_median(4)
_med48 = _make_median(3)


def _mm1(x, w):
    mrows, k = x.shape
    c = w.shape[1]
    bm = 400

    def body(x_ref, w_ref, o_ref):
        o_ref[...] = lax.dot_general(
            x_ref[...], w_ref[...], (((1,), (0,)), ((), ())),
            preferred_element_type=jnp.float32,
            precision=lax.Precision.HIGHEST)
    return pl.pallas_call(
        body,
        grid=(mrows // bm,),
        in_specs=[pl.BlockSpec((bm, k), lambda i: (i, 0)),
                  pl.BlockSpec((k, c), lambda i: (0, 0))],
        out_specs=pl.BlockSpec((bm, c), lambda i: (i, 0)),
        out_shape=jax.ShapeDtypeStruct((mrows, c), jnp.float32))(x, w)


def _mm2(m1, b1, w2p):
    bm = 400
    k = m1.shape[1]
    c = w2p.shape[1]

    def body(m_ref, b_ref, w_ref, o_ref):
        h = jnp.maximum(m_ref[...] + b_ref[...], 0.0)
        o_ref[...] = lax.dot_general(
            h, w_ref[...], (((1,), (0,)), ((), ())),
            preferred_element_type=jnp.float32,
            precision=lax.Precision.HIGHEST)
    return pl.pallas_call(
        body,
        grid=(N // bm,),
        in_specs=[pl.BlockSpec((bm, k), lambda i: (i, 0)),
                  pl.BlockSpec((1, k), lambda i: (0, 0)),
                  pl.BlockSpec((k, c), lambda i: (0, 0))],
        out_specs=pl.BlockSpec((bm, c), lambda i: (i, 0)),
        out_shape=jax.ShapeDtypeStruct((N, c), jnp.float32))(m1, b1, w2p)


def _final(m2, b2):
    bm = 400
    cp = m2.shape[1]

    def body(m_ref, b_ref, o_ref):
        y = m_ref[:, :40] + b_ref[...]
        mx = jnp.max(y, axis=1, keepdims=True)
        s = jnp.sum(jnp.exp(y - mx), axis=1, keepdims=True)
        o_ref[...] = y - mx - jnp.log(s)
    return pl.pallas_call(
        body,
        grid=(N // bm,),
        in_specs=[pl.BlockSpec((bm, cp), lambda i: (i, 0)),
                  pl.BlockSpec((1, 40), lambda i: (0, 0))],
        out_specs=pl.BlockSpec((bm, 40), lambda i: (i, 0)),
        out_shape=jax.ShapeDtypeStruct((N, 40), jnp.float32))(m2, b2)


def kernel(x, edge_index, W1, b1, W2, b2):
    loop = jnp.arange(N, dtype=edge_index.dtype)
    src2 = jnp.concatenate([edge_index[0], loop])
    dst2 = jnp.concatenate([edge_index[1], loop])

    h1 = _mm1(x, W1)
    counts, scd, scs = _a1(dst2, src2)
    starts, srcs = _a2(counts, scd, scs)
    m1 = _med64(h1, srcs, counts, starts)

    w2p = jnp.concatenate([W2, jnp.zeros((W2.shape[0], 8), jnp.float32)], axis=1)
    h2 = _mm2(m1, b1.reshape(1, -1), w2p)
    m2 = _med48(h2, srcs, counts, starts)
    return _final(m2, b2.reshape(1, -1))


# speedup vs baseline: 9.3149x; 1.0167x over previous
import functools

import numpy as np
import jax
import jax.numpy as jnp
from jax import lax
from jax.experimental import pallas as pl
from jax.experimental.pallas import tpu as pltpu
from jax.experimental.pallas import tpu_sc as plsc

N = 10000
E2 = 170000
NW = 32
NPT = 320
NPAD = NW * NPT
CHUNK = 2000
NCH = E2 // CHUNK
E2P = 84 * 2048
STG = 160
WCAP = 512
CSTEP = 504
MSBI = np.int32(-2147483648)
LOWI = np.int32(0x7FFFFFFF)

_MESH = plsc.VectorSubcoreMesh(core_axis_name="c", subcore_axis_name="s")


def _wid():
    return lax.axis_index("s") * 2 + lax.axis_index("c")


def _iota16():
    return lax.iota(jnp.int32, 16)


def _sread(ref, i):
    return ref[pl.ds(i, 16)][0]


@functools.partial(
    pl.kernel,
    out_type=[
        jax.ShapeDtypeStruct((NPAD,), jnp.int32),
        jax.ShapeDtypeStruct((NW, E2P), jnp.int32),
        jax.ShapeDtypeStruct((NW, E2P), jnp.int32),
    ],
    mesh=_MESH,
    compiler_params=pltpu.CompilerParams(needs_layout_passes=False),
    scratch_types=[
        pltpu.VMEM((CHUNK,), jnp.int32),
        pltpu.VMEM((CHUNK,), jnp.int32),
        pltpu.VMEM((STG,), jnp.int32),
        pltpu.VMEM((STG,), jnp.int32),
        pltpu.VMEM((NPT,), jnp.int32),
    ],
)
def _a1(dst_hbm, src_hbm, counts_hbm, scd_hbm, scs_hbm,
        dch, sch, sd, ss, cnts):
    wid = _wid()
    lo = wid * NPT
    z16 = jnp.zeros((16,), jnp.int32)

    def z1(i, _):
        cnts[pl.ds(i * 16, 16)] = z16
        return 0
    lax.fori_loop(0, NPT // 16, z1, 0)

    def flush128(fl):
        fo = pl.multiple_of(fl * 128, 8)
        pltpu.sync_copy(sd.at[pl.ds(0, 128)], scd_hbm.at[wid, pl.ds(fo, 128)])
        pltpu.sync_copy(ss.at[pl.ds(0, 128)], scs_hbm.at[wid, pl.ds(fo, 128)])
        td = sd[pl.ds(128, 16)]
        ts = ss[pl.ds(128, 16)]
        sd[pl.ds(0, 16)] = td
        ss[pl.ds(0, 16)] = ts

    def chunk_body(ci, carry):
        bp, fl = carry
        co = pl.multiple_of(ci * CHUNK, 8)
        pltpu.sync_copy(dst_hbm.at[pl.ds(co, CHUNK)], dch)
        pltpu.sync_copy(src_hbm.at[pl.ds(co, CHUNK)], sch)

        def g_body(g, carry):
            bp, fl = carry
            dv = dch[pl.ds(g * 16, 16)]
            sv = sch[pl.ds(g * 16, 16)]
            m = (dv >= lo) & (dv < lo + NPT)
            nn = jnp.clip(dv - lo, 0, NPT - 1)
            occ, lastm = plsc.scan_count(nn, mask=m)
            plsc.addupdate_scatter(cnts, [nn], occ, mask=m & lastm)
            plsc.store_compressed(sd.at[pl.ds(bp, 16)], dv, mask=m)
            plsc.store_compressed(ss.at[pl.ds(bp, 16)], sv, mask=m)
            bp = bp + jnp.sum(m.astype(jnp.int32))

            def do_flush(carry):
                bp, fl = carry
                flush128(fl)
                return bp - 128, fl + 1
            return lax.cond(bp >= 128, do_flush, lambda c: c, (bp, fl))
        return lax.fori_loop(0, CHUNK // 16, g_body, (bp, fl))
    bp, fl = lax.fori_loop(0, NCH, chunk_body, (0, 0))
    flush128(fl)
    flush128(fl + 1)
    pltpu.sync_copy(cnts, counts_hbm.at[pl.ds(lo, NPT)])


@functools.partial(
    pl.kernel,
    out_type=[
        jax.ShapeDtypeStruct((NPAD,), jnp.int32),
        jax.ShapeDtypeStruct((E2 + 8,), jnp.int32),
    ],
    mesh=_MESH,
    compiler_params=pltpu.CompilerParams(needs_layout_passes=False),
    scratch_types=[
        pltpu.VMEM((2048,), jnp.int32),
        pltpu.VMEM((2048,), jnp.int32),
        pltpu.VMEM((NPAD,), jnp.int32),
        pltpu.VMEM((NPT,), jnp.int32),
        pltpu.VMEM((NPT,), jnp.int32),
        pltpu.VMEM((STG,), jnp.int32),
        pltpu.VMEM((STG,), jnp.int32),
        pltpu.VMEM((128,), jnp.int32),
        pltpu.VMEM((128,), jnp.int32),
        pltpu.SemaphoreType.DMA,
    ],
)
def _a2(counts_hbm, scd_hbm, scs_hbm, starts_hbm, srcs_hbm,
        dch, sch, cbuf, stb, run, sp, sv, posb, valb, sem):
    wid = _wid()
    lo = wid * NPT
    pltpu.sync_copy(counts_hbm, cbuf)

    def ps(t, b):
        return b + jnp.sum(cbuf[pl.ds(t * 16, 16)])
    base = lax.fori_loop(0, wid * (NPT // 16), ps, 0)

    def sc_body(t, s):
        g = cbuf[pl.ds(lo + t * 16, 16)]
        excl = plsc.cumsum(g) - g + s
        stb[pl.ds(t * 16, 16)] = excl
        run[pl.ds(t * 16, 16)] = excl
        return s + jnp.sum(g)
    total = lax.fori_loop(0, NPT // 16, sc_body, base)
    local_e = total - base
    pltpu.sync_copy(stb, starts_hbm.at[pl.ds(lo, NPT)])

    def flush128(fl):
        for q in range(8):
            posb[pl.ds(q * 16, 16)] = sp[pl.ds(q * 16, 16)]
            valb[pl.ds(q * 16, 16)] = sv[pl.ds(q * 16, 16)]
        pltpu.async_copy(valb, srcs_hbm.at[posb], sem).wait()
        tp = sp[pl.ds(128, 16)]
        tv = sv[pl.ds(128, 16)]
        sp[pl.ds(0, 16)] = tp
        sv[pl.ds(0, 16)] = tv

    nchk = (local_e + 2047) // 2048

    def chunk_body(ci, carry):
        bp = carry
        co = pl.multiple_of(ci * 2048, 8)
        pltpu.sync_copy(scd_hbm.at[wid, pl.ds(co, 2048)], dch)
        pltpu.sync_copy(scs_hbm.at[wid, pl.ds(co, 2048)], sch)

        def g_body(g, bp):
            dv = dch[pl.ds(g * 16, 16)]
            s_v = sch[pl.ds(g * 16, 16)]
            valid = (ci * 2048 + g * 16 + _iota16()) < local_e
            nn = jnp.clip(dv - lo, 0, NPT - 1)
            occ, lastm = plsc.scan_count(nn, mask=valid)
            bs = plsc.load_gather(run, [nn], mask=valid)
            pos = bs + occ - 1
            plsc.addupdate_scatter(run, [nn], occ, mask=valid & lastm)
            plsc.store_compressed(sp.at[pl.ds(bp, 16)], pos, mask=valid)
            plsc.store_compressed(sv.at[pl.ds(bp, 16)], s_v, mask=valid)
            bp = bp + jnp.sum(valid.astype(jnp.int32))

            def do_flush(bp):
                flush128(0)
                return bp - 128
            return lax.cond(bp >= 128, do_flush, lambda b: b, bp)
        return lax.fori_loop(0, 128, g_body, bp)
    bp = lax.fori_loop(0, nchk, chunk_body, 0)

    def pad_and_flush(bp):
        for q in range(STG // 16):
            lane = _iota16() + q * 16
            pv = sp[pl.ds(q * 16, 16)]
            sp[pl.ds(q * 16, 16)] = jnp.where(lane < bp, pv, E2)
        flush128(0)
        return jnp.maximum(bp - 128, 0)
    bp = pad_and_flush(bp)
    pad_and_flush(bp)


def _make_median(ngrp):
    C = ngrp * 16

    def gather_convert(h_hbm, srcs_hbm, idx4, rb, ub, sem, base_al):
        base_al = pl.multiple_of(base_al, 8)
        for kk in range(4):
            pltpu.sync_copy(srcs_hbm.at[pl.ds(base_al + kk * 128, 128)], idx4[kk])
        descs = [
            pltpu.async_copy(h_hbm.at[idx4[kk]], rb.at[pl.ds(kk * 128, 128)], sem)
            for kk in range(4)]
        for dsc in descs:
            dsc.wait()

        def cv(e, _):
            for g in range(ngrp):
                v = rb[e, pl.ds(g * 16, 16)]
                b = plsc.bitcast(v, jnp.int32)
                m = b >> 31
                ub[e, pl.ds(g * 16, 16)] = b ^ (m & LOWI)
            return 0
        lax.fori_loop(0, WCAP, cv, 0, unroll=4)

    def make_ts(P, t):
        sh = 30 - 2 * t
        t1 = tuple(P[g] | (np.int32(1) << sh) for g in range(ngrp))
        t2 = tuple(P[g] | (np.int32(2) << sh) for g in range(ngrp))
        t3 = tuple(P[g] | (np.int32(3) << sh) for g in range(ngrp))
        tc = tuple((t1[g] ^ MSBI, t2[g] ^ MSBI, t3[g] ^ MSBI)
                   for g in range(ngrp))
        return t1, t2, t3, tc

    def select_update(P, cnts, t1, t2, t3, k1):
        out = []
        for g in range(ngrp):
            c1, c2, c3 = cnts[3 * g], cnts[3 * g + 1], cnts[3 * g + 2]
            sel = jnp.where(c2 >= k1, t1[g], jnp.where(c3 >= k1, t2[g], t3[g]))
            out.append(jnp.where(c1 >= k1, P[g], sel))
        return tuple(out)

    def count_seg(ub, e0, d, tc, cnts):
        def e_body(e, cn):
            row = e0 + e
            new = []
            for g in range(ngrp):
                u = ub[row, pl.ds(g * 16, 16)]
                new.append(cn[3 * g] + (u < tc[g][0]).astype(jnp.int32))
                new.append(cn[3 * g + 1] + (u < tc[g][1]).astype(jnp.int32))
                new.append(cn[3 * g + 2] + (u < tc[g][2]).astype(jnp.int32))
            return tuple(new)
        return lax.fori_loop(0, d, e_body, cnts)

    def finish(P, outb, i):
        for g in range(ngrp):
            ui = P[g]
            b = jnp.where(ui < 0, ui ^ MSBI, ~ui)
            outb[i, pl.ds(g * 16, 16)] = plsc.bitcast(b, jnp.float32)

    @functools.partial(
        pl.kernel,
        out_type=jax.ShapeDtypeStruct((NPAD, C), jnp.float32),
        mesh=_MESH,
        compiler_params=pltpu.CompilerParams(
            needs_layout_passes=False, use_tc_tiling_on_sc=False),
        scratch_types=[
            pltpu.VMEM((NPT + 16,), jnp.int32),
            pltpu.VMEM((NPT + 16,), jnp.int32),
            pltpu.VMEM((128,), jnp.int32),
            pltpu.VMEM((128,), jnp.int32),
            pltpu.VMEM((128,), jnp.int32),
            pltpu.VMEM((128,), jnp.int32),
            pltpu.VMEM((WCAP, C), jnp.float32),
            pltpu.VMEM((WCAP, C), jnp.int32),
            pltpu.VMEM((NPT, C), jnp.float32),
            pltpu.SemaphoreType.DMA,
        ],
    )
    def med(h_hbm, srcs_hbm, counts_hbm, starts_hbm, m_hbm,
            cbuf, stb, i0, i1, i2, i3, rb, ub, outb, sem):
        wid = _wid()
        lo = wid * NPT
        idx4 = (i0, i1, i2, i3)
        pltpu.sync_copy(counts_hbm.at[pl.ds(lo, NPT)], cbuf.at[pl.ds(0, NPT)])
        pltpu.sync_copy(starts_hbm.at[pl.ds(lo, NPT)], stb.at[pl.ds(0, NPT)])
        fz16 = jnp.zeros((16,), jnp.float32)

        def zb(i, _):
            for g in range(ngrp):
                outb[i, pl.ds(g * 16, 16)] = fz16
            return 0
        lax.fori_loop(0, NPT, zb, 0)

        zP = tuple(jnp.zeros((16,), jnp.int32) for _ in range(ngrp))
        zC = tuple(jnp.zeros((16,), jnp.int32) for _ in range(3 * ngrp))

        def win_body(n):
            d = _sread(cbuf, n)
            ws = _sread(stb, n)
            ws_al = jnp.minimum(ws - lax.rem(ws, 8), E2 - WCAP)
            in_window = (ws + d) <= (ws_al + WCAP)

            def dowin(n):
                limit = ws_al + WCAP

                def ext_body(carry):
                    mm, _ = carry
                    ok = (mm < NPT) & (_sread(stb, mm) + _sread(cbuf, mm) <= limit)
                    return jnp.where(ok, mm + 1, mm), ~ok
                m, _ = lax.while_loop(lambda c: ~c[1], ext_body, (n + 1, False))
                gather_convert(h_hbm, srcs_hbm, idx4, rb, ub, sem, ws_al)

                def node_body(i, _):
                    di = _sread(cbuf, i)

                    def comp(_):
                        k1 = (di - 1) // 2 + 1
                        e0 = _sread(stb, i) - ws_al

                        def bit_body(t, P):
                            t1, t2, t3, tc = make_ts(P, t)
                            cnts = count_seg(ub, e0, di, tc, zC)
                            return select_update(P, cnts, t1, t2, t3, k1)
                        P = lax.fori_loop(0, 16, bit_body, zP)
                        finish(P, outb, i)
                        return 0
                    return lax.cond(di > 0, comp, lambda _: 0, 0)
                lax.fori_loop(n, m, node_body, 0)
                return m

            def dobig(n):
                k1 = (d - 1) // 2 + 1
                nchk = (d + CSTEP - 1) // CSTEP

                def bit_body(t, P):
                    t1, t2, t3, tc = make_ts(P, t)

                    def chunk_body(c2, cn):
                        es = ws + c2 * CSTEP
                        es_al = jnp.minimum(es - lax.rem(es, 8), E2 - WCAP)
                        gather_convert(h_hbm, srcs_hbm, idx4, rb, ub, sem, es_al)
                        cl = jnp.minimum(CSTEP, d - c2 * CSTEP)
                        return count_seg(ub, es - es_al, cl, tc, cn)
                    cnts = lax.fori_loop(0, nchk, chunk_body, zC)
                    return select_update(P, cnts, t1, t2, t3, k1)
                P = lax.fori_loop(0, 16, bit_body, zP)
                finish(P, outb, n)
                return n + 1

            def nonzero(n):
                return lax.cond(in_window, dowin, dobig, n)
            return lax.cond(d == 0, lambda v: v + 1, nonzero, n)
        lax.while_loop(lambda n: n < NPT, win_body, 0)
        pltpu.sync_copy(outb, m_hbm.at[pl.ds(lo, NPT)])

    return med


_med64 = _make_median(4)
_med48 = _make_median(3)


def _mm1(x, w):
    mrows, k = x.shape
    c = w.shape[1]
    bm = 400

    def body(x_ref, w_ref, o_ref):
        o_ref[...] = lax.dot_general(
            x_ref[...], w_ref[...], (((1,), (0,)), ((), ())),
            preferred_element_type=jnp.float32,
            precision=lax.Precision.HIGHEST)
    return pl.pallas_call(
        body,
        grid=(mrows // bm,),
        in_specs=[pl.BlockSpec((bm, k), lambda i: (i, 0)),
                  pl.BlockSpec((k, c), lambda i: (0, 0))],
        out_specs=pl.BlockSpec((bm, c), lambda i: (i, 0)),
        out_shape=jax.ShapeDtypeStruct((mrows, c), jnp.float32))(x, w)


def _mm2(m1, b1, w2p):
    bm = 400
    k = m1.shape[1]
    c = w2p.shape[1]

    def body(m_ref, b_ref, w_ref, o_ref):
        h = jnp.maximum(m_ref[...] + b_ref[...], 0.0)
        o_ref[...] = lax.dot_general(
            h, w_ref[...], (((1,), (0,)), ((), ())),
            preferred_element_type=jnp.float32,
            precision=lax.Precision.HIGHEST)
    return pl.pallas_call(
        body,
        grid=(N // bm,),
        in_specs=[pl.BlockSpec((bm, k), lambda i: (i, 0)),
                  pl.BlockSpec((1, k), lambda i: (0, 0)),
                  pl.BlockSpec((k, c), lambda i: (0, 0))],
        out_specs=pl.BlockSpec((bm, c), lambda i: (i, 0)),
        out_shape=jax.ShapeDtypeStruct((N, c), jnp.float32))(m1, b1, w2p)


def _final(m2, b2):
    bm = 400
    cp = m2.shape[1]

    def body(m_ref, b_ref, o_ref):
        y = m_ref[:, :40] + b_ref[...]
        mx = jnp.max(y, axis=1, keepdims=True)
        s = jnp.sum(jnp.exp(y - mx), axis=1, keepdims=True)
        o_ref[...] = y - mx - jnp.log(s)
    return pl.pallas_call(
        body,
        grid=(N // bm,),
        in_specs=[pl.BlockSpec((bm, cp), lambda i: (i, 0)),
                  pl.BlockSpec((1, 40), lambda i: (0, 0))],
        out_specs=pl.BlockSpec((bm, 40), lambda i: (i, 0)),
        out_shape=jax.ShapeDtypeStruct((N, 40), jnp.float32))(m2, b2)


def kernel(x, edge_index, W1, b1, W2, b2):
    loop = jnp.arange(N, dtype=edge_index.dtype)
    src2 = jnp.concatenate([edge_index[0], loop])
    dst2 = jnp.concatenate([edge_index[1], loop])

    h1 = _mm1(x, W1)
    counts, scd, scs = _a1(dst2, src2)
    starts, srcs = _a2(counts, scd, scs)
    m1 = _med64(h1, srcs, counts, starts)

    w2p = jnp.concatenate([W2, jnp.zeros((W2.shape[0], 8), jnp.float32)], axis=1)
    h2 = _mm2(m1, b1.reshape(1, -1), w2p)
    m2 = _med48(h2, srcs, counts, starts)
    return _final(m2, b2.reshape(1, -1))


# speedup vs baseline: 13.3255x; 1.4306x over previous
import functools

import numpy as np
import jax
import jax.numpy as jnp
from jax import lax
from jax.experimental import pallas as pl
from jax.experimental.pallas import tpu as pltpu
from jax.experimental.pallas import tpu_sc as plsc

N = 10000
E2 = 170000
NW = 32
NPT = 320
NPAD = NW * NPT
CHUNK = 2000
NCH = E2 // CHUNK
E2P = 84 * 2048
STG = 160
WCAP = 512
CSTEP = 504
MSBI = np.int32(-2147483648)
LOWI = np.int32(0x7FFFFFFF)

def _batcher_pairs(n):
    pairs = []

    def merge(lo, n2, r):
        step = r * 2
        if step < n2:
            merge(lo, n2, step)
            merge(lo + r, n2, step)
            for i in range(lo + r, lo + n2 - r, step):
                pairs.append((i, i + r))
        else:
            pairs.append((lo, lo + r))

    def sort(lo, n2):
        if n2 > 1:
            m = n2 // 2
            sort(lo, m)
            sort(lo + m, m)
            merge(lo, n2, 1)
    sort(0, n)
    return pairs


_PAIRS32 = _batcher_pairs(32)

_MESH = plsc.VectorSubcoreMesh(core_axis_name="c", subcore_axis_name="s")


def _wid():
    return lax.axis_index("s") * 2 + lax.axis_index("c")


def _iota16():
    return lax.iota(jnp.int32, 16)


def _sread(ref, i):
    return ref[pl.ds(i, 16)][0]


@functools.partial(
    pl.kernel,
    out_type=[
        jax.ShapeDtypeStruct((NPAD,), jnp.int32),
        jax.ShapeDtypeStruct((NW, E2P), jnp.int32),
        jax.ShapeDtypeStruct((NW, E2P), jnp.int32),
    ],
    mesh=_MESH,
    compiler_params=pltpu.CompilerParams(needs_layout_passes=False),
    scratch_types=[
        pltpu.VMEM((CHUNK,), jnp.int32),
        pltpu.VMEM((CHUNK,), jnp.int32),
        pltpu.VMEM((STG,), jnp.int32),
        pltpu.VMEM((STG,), jnp.int32),
        pltpu.VMEM((NPT,), jnp.int32),
    ],
)
def _a1(dst_hbm, src_hbm, counts_hbm, scd_hbm, scs_hbm,
        dch, sch, sd, ss, cnts):
    wid = _wid()
    lo = wid * NPT
    z16 = jnp.zeros((16,), jnp.int32)

    def z1(i, _):
        cnts[pl.ds(i * 16, 16)] = z16
        return 0
    lax.fori_loop(0, NPT // 16, z1, 0)

    def flush128(fl):
        fo = pl.multiple_of(fl * 128, 8)
        pltpu.sync_copy(sd.at[pl.ds(0, 128)], scd_hbm.at[wid, pl.ds(fo, 128)])
        pltpu.sync_copy(ss.at[pl.ds(0, 128)], scs_hbm.at[wid, pl.ds(fo, 128)])
        td = sd[pl.ds(128, 16)]
        ts = ss[pl.ds(128, 16)]
        sd[pl.ds(0, 16)] = td
        ss[pl.ds(0, 16)] = ts

    def chunk_body(ci, carry):
        bp, fl = carry
        co = pl.multiple_of(ci * CHUNK, 8)
        pltpu.sync_copy(dst_hbm.at[pl.ds(co, CHUNK)], dch)
        pltpu.sync_copy(src_hbm.at[pl.ds(co, CHUNK)], sch)

        def g_body(g, carry):
            bp, fl = carry
            dv = dch[pl.ds(g * 16, 16)]
            sv = sch[pl.ds(g * 16, 16)]
            m = (dv >= lo) & (dv < lo + NPT)
            nn = jnp.clip(dv - lo, 0, NPT - 1)
            occ, lastm = plsc.scan_count(nn, mask=m)
            plsc.addupdate_scatter(cnts, [nn], occ, mask=m & lastm)
            plsc.store_compressed(sd.at[pl.ds(bp, 16)], dv, mask=m)
            plsc.store_compressed(ss.at[pl.ds(bp, 16)], sv, mask=m)
            bp = bp + jnp.sum(m.astype(jnp.int32))

            def do_flush(carry):
                bp, fl = carry
                flush128(fl)
                return bp - 128, fl + 1
            return lax.cond(bp >= 128, do_flush, lambda c: c, (bp, fl))
        return lax.fori_loop(0, CHUNK // 16, g_body, (bp, fl))
    bp, fl = lax.fori_loop(0, NCH, chunk_body, (0, 0))
    flush128(fl)
    flush128(fl + 1)
    pltpu.sync_copy(cnts, counts_hbm.at[pl.ds(lo, NPT)])


@functools.partial(
    pl.kernel,
    out_type=[
        jax.ShapeDtypeStruct((NPAD,), jnp.int32),
        jax.ShapeDtypeStruct((E2 + 8,), jnp.int32),
    ],
    mesh=_MESH,
    compiler_params=pltpu.CompilerParams(needs_layout_passes=False),
    scratch_types=[
        pltpu.VMEM((2048,), jnp.int32),
        pltpu.VMEM((2048,), jnp.int32),
        pltpu.VMEM((NPT,), jnp.int32),
        pltpu.VMEM((NPT,), jnp.int32),
        pltpu.VMEM((NPT,), jnp.int32),
        pltpu.VMEM((STG,), jnp.int32),
        pltpu.VMEM((STG,), jnp.int32),
        pltpu.VMEM((128,), jnp.int32),
        pltpu.VMEM((128,), jnp.int32),
        pltpu.SemaphoreType.DMA,
    ],
)
def _a2(counts_hbm, scd_hbm, scs_hbm, starts_hbm, srcs_hbm,
        dch, sch, cbuf, stb, run, sp, sv, posb, valb, sem):
    wid = _wid()
    lo = wid * NPT

    def pw(w2, base):
        pltpu.sync_copy(counts_hbm.at[pl.ds(pl.multiple_of(w2 * NPT, 8), NPT)], cbuf)

        def ps(t, b):
            return b + jnp.sum(cbuf[pl.ds(t * 16, 16)])
        return lax.fori_loop(0, NPT // 16, ps, base)
    base = lax.fori_loop(0, wid, pw, 0)

    pltpu.sync_copy(counts_hbm.at[pl.ds(lo, NPT)], cbuf)

    def sc_body(t, s):
        g = cbuf[pl.ds(t * 16, 16)]
        excl = plsc.cumsum(g) - g + s
        stb[pl.ds(t * 16, 16)] = excl
        run[pl.ds(t * 16, 16)] = excl
        return s + jnp.sum(g)
    total = lax.fori_loop(0, NPT // 16, sc_body, base)
    local_e = total - base
    pltpu.sync_copy(stb, starts_hbm.at[pl.ds(lo, NPT)])

    def flush128(fl):
        for q in range(8):
            posb[pl.ds(q * 16, 16)] = sp[pl.ds(q * 16, 16)]
            valb[pl.ds(q * 16, 16)] = sv[pl.ds(q * 16, 16)]
        pltpu.async_copy(valb, srcs_hbm.at[posb], sem).wait()
        tp = sp[pl.ds(128, 16)]
        tv = sv[pl.ds(128, 16)]
        sp[pl.ds(0, 16)] = tp
        sv[pl.ds(0, 16)] = tv

    nchk = (local_e + 2047) // 2048

    def chunk_body(ci, carry):
        bp = carry
        co = pl.multiple_of(ci * 2048, 8)
        pltpu.sync_copy(scd_hbm.at[wid, pl.ds(co, 2048)], dch)
        pltpu.sync_copy(scs_hbm.at[wid, pl.ds(co, 2048)], sch)

        def g_body(g, bp):
            dv = dch[pl.ds(g * 16, 16)]
            s_v = sch[pl.ds(g * 16, 16)]
            valid = (ci * 2048 + g * 16 + _iota16()) < local_e
            nn = jnp.clip(dv - lo, 0, NPT - 1)
            occ, lastm = plsc.scan_count(nn, mask=valid)
            bs = plsc.load_gather(run, [nn], mask=valid)
            pos = bs + occ - 1
            plsc.addupdate_scatter(run, [nn], occ, mask=valid & lastm)
            plsc.store_compressed(sp.at[pl.ds(bp, 16)], pos, mask=valid)
            plsc.store_compressed(sv.at[pl.ds(bp, 16)], s_v, mask=valid)
            bp = bp + jnp.sum(valid.astype(jnp.int32))

            def do_flush(bp):
                flush128(0)
                return bp - 128
            return lax.cond(bp >= 128, do_flush, lambda b: b, bp)
        return lax.fori_loop(0, 128, g_body, bp)
    bp = lax.fori_loop(0, nchk, chunk_body, 0)

    def pad_and_flush(bp):
        for q in range(STG // 16):
            lane = _iota16() + q * 16
            pv = sp[pl.ds(q * 16, 16)]
            sp[pl.ds(q * 16, 16)] = jnp.where(lane < bp, pv, E2)
        flush128(0)
        return jnp.maximum(bp - 128, 0)
    bp = pad_and_flush(bp)
    pad_and_flush(bp)


def _make_median(ngrp):
    C = ngrp * 16

    def gather_convert(h_hbm, srcs_hbm, idx4, rb, ub, sem, base_al):
        base_al = pl.multiple_of(base_al, 8)
        for kk in range(4):
            pltpu.sync_copy(srcs_hbm.at[pl.ds(base_al + kk * 128, 128)], idx4[kk])
        descs = [
            pltpu.async_copy(h_hbm.at[idx4[kk]], rb.at[pl.ds(kk * 128, 128)], sem)
            for kk in range(4)]
        for dsc in descs:
            dsc.wait()

        def cv(e, _):
            for g in range(ngrp):
                v = rb[e, pl.ds(g * 16, 16)]
                b = plsc.bitcast(v, jnp.int32)
                m = b >> 31
                ub[e, pl.ds(g * 16, 16)] = b ^ (m & LOWI)
            return 0
        lax.fori_loop(0, WCAP, cv, 0, unroll=4)

    def make_ts(P, t):
        sh = 30 - 2 * t
        t1 = tuple(P[g] | (np.int32(1) << sh) for g in range(ngrp))
        t2 = tuple(P[g] | (np.int32(2) << sh) for g in range(ngrp))
        t3 = tuple(P[g] | (np.int32(3) << sh) for g in range(ngrp))
        tc = tuple((t1[g] ^ MSBI, t2[g] ^ MSBI, t3[g] ^ MSBI)
                   for g in range(ngrp))
        return t1, t2, t3, tc

    def select_update(P, cnts, t1, t2, t3, k1):
        out = []
        for g in range(ngrp):
            c1, c2, c3 = cnts[3 * g], cnts[3 * g + 1], cnts[3 * g + 2]
            sel = jnp.where(c2 >= k1, t1[g], jnp.where(c3 >= k1, t2[g], t3[g]))
            out.append(jnp.where(c1 >= k1, P[g], sel))
        return tuple(out)

    def count_seg(ub, e0, d, tc, cnts):
        def e_body(e, cn):
            row = e0 + e
            new = []
            for g in range(ngrp):
                u = ub[row, pl.ds(g * 16, 16)]
                new.append(cn[3 * g] + (u < tc[g][0]).astype(jnp.int32))
                new.append(cn[3 * g + 1] + (u < tc[g][1]).astype(jnp.int32))
                new.append(cn[3 * g + 2] + (u < tc[g][2]).astype(jnp.int32))
            return tuple(new)
        return lax.fori_loop(0, d, e_body, cnts)

    def finish(P, outb, i):
        for g in range(ngrp):
            ui = P[g]
            b = jnp.where(ui < 0, ui ^ MSBI, ~ui)
            outb[i, pl.ds(g * 16, 16)] = plsc.bitcast(b, jnp.float32)

    @functools.partial(
        pl.kernel,
        out_type=jax.ShapeDtypeStruct((NPAD, C), jnp.float32),
        mesh=_MESH,
        compiler_params=pltpu.CompilerParams(
            needs_layout_passes=False, use_tc_tiling_on_sc=False),
        scratch_types=[
            pltpu.VMEM((NPT + 16,), jnp.int32),
            pltpu.VMEM((NPT + 16,), jnp.int32),
            pltpu.VMEM((128,), jnp.int32),
            pltpu.VMEM((128,), jnp.int32),
            pltpu.VMEM((128,), jnp.int32),
            pltpu.VMEM((128,), jnp.int32),
            pltpu.VMEM((WCAP, C), jnp.float32),
            pltpu.VMEM((WCAP, C), jnp.int32),
            pltpu.VMEM((NPT, C), jnp.float32),
            pltpu.SemaphoreType.DMA,
        ],
    )
    def med(h_hbm, srcs_hbm, counts_hbm, starts_hbm, m_hbm,
            cbuf, stb, i0, i1, i2, i3, rb, ub, outb, sem):
        wid = _wid()
        lo = wid * NPT
        idx4 = (i0, i1, i2, i3)
        pltpu.sync_copy(counts_hbm.at[pl.ds(lo, NPT)], cbuf.at[pl.ds(0, NPT)])
        pltpu.sync_copy(starts_hbm.at[pl.ds(lo, NPT)], stb.at[pl.ds(0, NPT)])
        fz16 = jnp.zeros((16,), jnp.float32)

        def zb(i, _):
            for g in range(ngrp):
                outb[i, pl.ds(g * 16, 16)] = fz16
            return 0
        lax.fori_loop(0, NPT, zb, 0)

        zP = tuple(jnp.zeros((16,), jnp.int32) for _ in range(ngrp))
        zC = tuple(jnp.zeros((16,), jnp.int32) for _ in range(3 * ngrp))

        def win_body(n):
            d = _sread(cbuf, n)
            ws = _sread(stb, n)
            ws_al = jnp.minimum(ws - lax.rem(ws, 8), E2 - WCAP)
            in_window = (ws + d) <= (ws_al + WCAP)

            def dowin(n):
                limit = ws_al + WCAP

                def ext_body(carry):
                    mm, _ = carry
                    ok = (mm < NPT) & (_sread(stb, mm) + _sread(cbuf, mm) <= limit)
                    return jnp.where(ok, mm + 1, mm), ~ok
                m, _ = lax.while_loop(lambda c: ~c[1], ext_body, (n + 1, False))
                gather_convert(h_hbm, srcs_hbm, idx4, rb, ub, sem, ws_al)

                def node_body(i, _):
                    di = _sread(cbuf, i)

                    def comp_net(_):
                        kq = (di - 1) // 2
                        e0 = _sread(stb, i) - ws_al
                        for g in range(ngrp):
                            vs = []
                            for j in range(32):
                                row = jnp.minimum(e0 + j, WCAP - 1)
                                v = rb[row, pl.ds(g * 16, 16)]
                                vs.append(jnp.where(j < di, v, jnp.float32(np.inf)))
                            for (a, b) in _PAIRS32:
                                xa, xb = vs[a], vs[b]
                                vs[a] = jnp.minimum(xa, xb)
                                vs[b] = jnp.maximum(xa, xb)
                            cur = vs[:16]
                            for lev in range(4):
                                bit = (kq >> lev) & 1
                                cur = [jnp.where(bit == 1, cur[2 * q + 1], cur[2 * q])
                                       for q in range(len(cur) // 2)]
                            outb[i, pl.ds(g * 16, 16)] = cur[0]
                        return 0

                    def comp(_):
                        k1 = (di - 1) // 2 + 1
                        e0 = _sread(stb, i) - ws_al

                        def bit_body(t, P):
                            t1, t2, t3, tc = make_ts(P, t)
                            cnts = count_seg(ub, e0, di, tc, zC)
                            return select_update(P, cnts, t1, t2, t3, k1)
                        P = lax.fori_loop(0, 16, bit_body, zP)
                        finish(P, outb, i)
                        return 0

                    def nz(_):
                        return lax.cond(di <= 32, comp_net, comp, 0)
                    return lax.cond(di > 0, nz, lambda _: 0, 0)
                lax.fori_loop(n, m, node_body, 0)
                return m

            def dobig(n):
                k1 = (d - 1) // 2 + 1
                nchk = (d + CSTEP - 1) // CSTEP

                def bit_body(t, P):
                    t1, t2, t3, tc = make_ts(P, t)

                    def chunk_body(c2, cn):
                        es = ws + c2 * CSTEP
                        es_al = jnp.minimum(es - lax.rem(es, 8), E2 - WCAP)
                        gather_convert(h_hbm, srcs_hbm, idx4, rb, ub, sem, es_al)
                        cl = jnp.minimum(CSTEP, d - c2 * CSTEP)
                        return count_seg(ub, es - es_al, cl, tc, cn)
                    cnts = lax.fori_loop(0, nchk, chunk_body, zC)
                    return select_update(P, cnts, t1, t2, t3, k1)
                P = lax.fori_loop(0, 16, bit_body, zP)
                finish(P, outb, n)
                return n + 1

            def nonzero(n):
                return lax.cond(in_window, dowin, dobig, n)
            return lax.cond(d == 0, lambda v: v + 1, nonzero, n)
        lax.while_loop(lambda n: n < NPT, win_body, 0)
        pltpu.sync_copy(outb, m_hbm.at[pl.ds(lo, NPT)])

    return med


_med64 = _make_median(4)
_med48 = _make_median(3)


def _mm1(x, w):
    mrows, k = x.shape
    c = w.shape[1]
    bm = 400

    def body(x_ref, w_ref, o_ref):
        o_ref[...] = lax.dot_general(
            x_ref[...], w_ref[...], (((1,), (0,)), ((), ())),
            preferred_element_type=jnp.float32,
            precision=lax.Precision.HIGHEST)
    return pl.pallas_call(
        body,
        grid=(mrows // bm,),
        in_specs=[pl.BlockSpec((bm, k), lambda i: (i, 0)),
                  pl.BlockSpec((k, c), lambda i: (0, 0))],
        out_specs=pl.BlockSpec((bm, c), lambda i: (i, 0)),
        out_shape=jax.ShapeDtypeStruct((mrows, c), jnp.float32))(x, w)


def _mm2(m1, b1, w2p):
    bm = 400
    k = m1.shape[1]
    c = w2p.shape[1]

    def body(m_ref, b_ref, w_ref, o_ref):
        h = jnp.maximum(m_ref[...] + b_ref[...], 0.0)
        o_ref[...] = lax.dot_general(
            h, w_ref[...], (((1,), (0,)), ((), ())),
            preferred_element_type=jnp.float32,
            precision=lax.Precision.HIGHEST)
    return pl.pallas_call(
        body,
        grid=(N // bm,),
        in_specs=[pl.BlockSpec((bm, k), lambda i: (i, 0)),
                  pl.BlockSpec((1, k), lambda i: (0, 0)),
                  pl.BlockSpec((k, c), lambda i: (0, 0))],
        out_specs=pl.BlockSpec((bm, c), lambda i: (i, 0)),
        out_shape=jax.ShapeDtypeStruct((N, c), jnp.float32))(m1, b1, w2p)


def _final(m2, b2):
    bm = 400
    cp = m2.shape[1]

    def body(m_ref, b_ref, o_ref):
        y = m_ref[:, :40] + b_ref[...]
        mx = jnp.max(y, axis=1, keepdims=True)
        s = jnp.sum(jnp.exp(y - mx), axis=1, keepdims=True)
        o_ref[...] = y - mx - jnp.log(s)
    return pl.pallas_call(
        body,
        grid=(N // bm,),
        in_specs=[pl.BlockSpec((bm, cp), lambda i: (i, 0)),
                  pl.BlockSpec((1, 40), lambda i: (0, 0))],
        out_specs=pl.BlockSpec((bm, 40), lambda i: (i, 0)),
        out_shape=jax.ShapeDtypeStruct((N, 40), jnp.float32))(m2, b2)


def kernel(x, edge_index, W1, b1, W2, b2):
    loop = jnp.arange(N, dtype=edge_index.dtype)
    src2 = jnp.concatenate([edge_index[0], loop])
    dst2 = jnp.concatenate([edge_index[1], loop])

    h1 = _mm1(x, W1)
    counts, scd, scs = _a1(dst2, src2)
    starts, srcs = _a2(counts, scd, scs)
    m1 = _med64(h1, srcs, counts, starts)

    w2p = jnp.concatenate([W2, jnp.zeros((W2.shape[0], 8), jnp.float32)], axis=1)
    h2 = _mm2(m1, b1.reshape(1, -1), w2p)
    m2 = _med48(h2, srcs, counts, starts)
    return _final(m2, b2.reshape(1, -1))
